# baseline (speedup 1.0000x reference)
import functools

import jax
import jax.numpy as jnp
from jax import lax
from jax.experimental import pallas as pl
from jax.experimental.pallas import tpu as pltpu
from jax.experimental.pallas import tpu_sc as plsc

BATCH = 16384
HIDDEN = 64
BLK = 128
NUM_WORKERS = 32
RPW = BATCH // NUM_WORKERS
GROUPS = RPW // 16
NBUF = 12
CSTG = 16 * HIDDEN


def _build():
    mesh = plsc.VectorSubcoreMesh(core_axis_name="c", subcore_axis_name="s")

    @functools.partial(
        pl.kernel,
        mesh=mesh,
        out_type=jax.ShapeDtypeStruct((BATCH * HIDDEN,), jnp.float32),
        scratch_types=[
            pltpu.VMEM((RPW,), jnp.int32),
            pltpu.VMEM((RPW,), jnp.int32),
            pltpu.VMEM((RPW,), jnp.int32),
            pltpu.VMEM((RPW,), jnp.int32),
            pltpu.VMEM((NBUF * HIDDEN, BLK), jnp.float32),
            pltpu.VMEM((2 * CSTG,), jnp.float32),
            pltpu.SemaphoreType.DMA,
            pltpu.SemaphoreType.DMA,
            pltpu.SemaphoreType.DMA,
            pltpu.SemaphoreType.DMA,
            pltpu.SemaphoreType.DMA,
            pltpu.SemaphoreType.DMA,
        ],
        compiler_params=pltpu.CompilerParams(needs_layout_passes=False),
    )
    def emb(sidx_hbm, pos_hbm, newf_hbm, did_hbm, tabt_hbm, out_hbm,
            sidx_v, pos_v, newf_v, did_v, ring_v, cstg_v,
            sem0, sem1, sem2, sem3, semout0, semout1):
        wid = lax.axis_index("s") * 2 + lax.axis_index("c")
        base = wid * RPW
        pltpu.sync_copy(sidx_hbm.at[pl.ds(base, RPW)], sidx_v)
        pltpu.sync_copy(pos_hbm.at[pl.ds(base, RPW)], pos_v)
        pltpu.sync_copy(newf_hbm.at[pl.ds(base, RPW)], newf_v)
        pltpu.sync_copy(did_hbm.at[pl.ds(base, RPW)], did_v)
        sems = (sem0, sem1, sem2, sem3)
        did0 = did_v[pl.ds(0, 16)][0]
        rows4 = [
            jax.lax.iota(jnp.int32, 16) + (16 * q) for q in range(HIDDEN // 16)
        ]

        def fire(sv, nv, dv, u):
            for k in range(4 * u, 4 * u + 4):
                sk = sv[k]
                lk = sk & 127
                ck = pl.multiple_of(sk - lk, BLK)
                s64 = pl.multiple_of(lax.rem(dv[k] - did0, NBUF) * HIDDEN,
                                     HIDDEN)

                @pl.when(nv[k] != 0)
                def _():
                    pltpu.async_copy(
                        tabt_hbm.at[:, pl.ds(ck, BLK)],
                        ring_v.at[pl.ds(s64, HIDDEN), :],
                        sems[u],
                    )

        semouts = (semout0, semout1)

        def drain_extract(sv, pv, nv, dv, u, cpar, par):
            for k in range(4 * u, 4 * u + 4):
                s64 = pl.multiple_of(lax.rem(dv[k] - did0, NBUF) * HIDDEN,
                                     HIDDEN)

                @pl.when(nv[k] != 0)
                def _():
                    pltpu.make_async_copy(
                        tabt_hbm.at[:, pl.ds(0, BLK)],
                        ring_v.at[pl.ds(s64, HIDDEN), :],
                        sems[u],
                    ).wait()
            for k in range(4 * u, 4 * u + 4):
                sk = sv[k]
                lk = sk & 127
                s64 = lax.rem(dv[k] - did0, NBUF) * HIDDEN
                lane_v = jnp.full((16,), lk, jnp.int32)
                kk = k & 15
                for q in range(HIDDEN // 16):
                    vals = plsc.load_gather(ring_v, [rows4[q] + s64, lane_v])
                    cstg_v[pl.ds(pl.multiple_of(cpar + kk * HIDDEN + 16 * q,
                                                16), 16)] = vals
                po = pl.multiple_of(pv[k] * HIDDEN, HIDDEN)
                pltpu.async_copy(
                    cstg_v.at[pl.ds(pl.multiple_of(cpar + kk * HIDDEN, HIDDEN),
                                    HIDDEN)],
                    out_hbm.at[pl.ds(po, HIDDEN)],
                    semouts[par],
                )

        def process_group(g, par):
            g16 = pl.multiple_of(g * 16, 16)
            sv = sidx_v[pl.ds(g16, 16)]
            pv = pos_v[pl.ds(g16, 16)]
            nv = newf_v[pl.ds(g16, 16)]
            dv = did_v[pl.ds(g16, 16)]
            cpar = par * CSTG
            fire(sv, nv, dv, 0)
            fire(sv, nv, dv, 1)
            fire(sv, nv, dv, 2)
            drain_extract(sv, pv, nv, dv, 0, cpar, par)
            fire(sv, nv, dv, 3)
            drain_extract(sv, pv, nv, dv, 1, cpar, par)
            drain_extract(sv, pv, nv, dv, 2, cpar, par)
            drain_extract(sv, pv, nv, dv, 3, cpar, par)

        def drain_out(par):
            pltpu.make_async_copy(
                out_hbm.at[pl.ds(0, CSTG)],
                cstg_v.at[pl.ds(par * CSTG, CSTG)],
                semouts[par],
            ).wait()

        def body(t, carry):
            @pl.when(t >= 1)
            def _():
                drain_out(0)

            process_group(2 * t, 0)

            @pl.when(t >= 1)
            def _():
                drain_out(1)

            process_group(2 * t + 1, 1)
            return carry

        lax.fori_loop(0, GROUPS // 2, body, 0)
        drain_out(0)
        drain_out(1)

    return emb


_EMB = _build()


def kernel(labels, embedding_table):
    idx32 = labels.astype(jnp.int32)
    order = jnp.argsort(idx32).astype(jnp.int32)
    sidx = jnp.take(idx32, order)
    blk = sidx >> 7
    first = (jnp.arange(BATCH, dtype=jnp.int32) % RPW) == 0
    shifted = jnp.concatenate(
        [jnp.ones((1,), jnp.bool_), blk[1:] != blk[:-1]]
    )
    newf = (first | shifted).astype(jnp.int32)
    did = jnp.cumsum(newf).astype(jnp.int32) - 1
    table_t = jnp.swapaxes(embedding_table, 0, 1)
    out1d = _EMB(sidx, order, newf, did, table_t)
    return out1d.reshape(BATCH, HIDDEN)


# speedup vs baseline: 1.0104x; 1.0104x over previous
import functools

import jax
import jax.numpy as jnp
from jax import lax
from jax.experimental import pallas as pl
from jax.experimental.pallas import tpu as pltpu
from jax.experimental.pallas import tpu_sc as plsc

BATCH = 16384
HIDDEN = 64
BLK = 128
NUM_WORKERS = 32
RPW = BATCH // NUM_WORKERS
GROUPS = RPW // 16
NBUF = 8
CSTG = 16 * HIDDEN


def _build():
    mesh = plsc.VectorSubcoreMesh(core_axis_name="c", subcore_axis_name="s")

    @functools.partial(
        pl.kernel,
        mesh=mesh,
        out_type=jax.ShapeDtypeStruct((BATCH * HIDDEN,), jnp.float32),
        scratch_types=[
            pltpu.VMEM((RPW,), jnp.int32),
            pltpu.VMEM((RPW,), jnp.int32),
            pltpu.VMEM((RPW,), jnp.int32),
            pltpu.VMEM((RPW,), jnp.int32),
            pltpu.VMEM((NBUF * HIDDEN, BLK), jnp.float32),
            pltpu.VMEM((2 * CSTG,), jnp.float32),
            pltpu.SemaphoreType.DMA,
            pltpu.SemaphoreType.DMA,
            pltpu.SemaphoreType.DMA,
            pltpu.SemaphoreType.DMA,
        ],
        compiler_params=pltpu.CompilerParams(needs_layout_passes=False),
    )
    def emb(sidx_hbm, pos_hbm, newf_hbm, did_hbm, tabt_hbm, out_hbm,
            sidx_v, pos_v, newf_v, did_v, ring_v, cstg_v,
            sem0, sem1, semout0, semout1):
        wid = lax.axis_index("s") * 2 + lax.axis_index("c")
        base = wid * RPW
        pltpu.sync_copy(sidx_hbm.at[pl.ds(base, RPW)], sidx_v)
        pltpu.sync_copy(pos_hbm.at[pl.ds(base, RPW)], pos_v)
        pltpu.sync_copy(newf_hbm.at[pl.ds(base, RPW)], newf_v)
        pltpu.sync_copy(did_hbm.at[pl.ds(base, RPW)], did_v)
        sems = (sem0, sem1)
        did0 = did_v[pl.ds(0, 16)][0]
        rows4 = [
            jax.lax.iota(jnp.int32, 16) + (16 * q) for q in range(HIDDEN // 16)
        ]

        def fire(sv, nv, dv, u):
            for k in range(4 * u, 4 * u + 4):
                sk = sv[k]
                lk = sk & 127
                ck = pl.multiple_of(sk - lk, BLK)
                s64 = pl.multiple_of(((dv[k] - did0) & (NBUF - 1)) * HIDDEN,
                                     HIDDEN)

                @pl.when(nv[k] != 0)
                def _():
                    pltpu.async_copy(
                        tabt_hbm.at[:, pl.ds(ck, BLK)],
                        ring_v.at[pl.ds(s64, HIDDEN), :],
                        sems[u % 2],
                    )

        semouts = (semout0, semout1)

        def drain_extract(sv, pv, nv, dv, u, cpar, par):
            for k in range(4 * u, 4 * u + 4):
                s64 = pl.multiple_of(((dv[k] - did0) & (NBUF - 1)) * HIDDEN,
                                     HIDDEN)

                @pl.when(nv[k] != 0)
                def _():
                    pltpu.make_async_copy(
                        tabt_hbm.at[:, pl.ds(0, BLK)],
                        ring_v.at[pl.ds(s64, HIDDEN), :],
                        sems[u % 2],
                    ).wait()
            for k in range(4 * u, 4 * u + 4):
                sk = sv[k]
                lk = sk & 127
                s64 = ((dv[k] - did0) & (NBUF - 1)) * HIDDEN
                lane_v = jnp.full((16,), lk, jnp.int32)
                kk = k & 15
                for q in range(HIDDEN // 16):
                    vals = plsc.load_gather(ring_v, [rows4[q] + s64, lane_v])
                    cstg_v[pl.ds(pl.multiple_of(cpar + kk * HIDDEN + 16 * q,
                                                16), 16)] = vals
                po = pl.multiple_of(pv[k] * HIDDEN, HIDDEN)
                pltpu.async_copy(
                    cstg_v.at[pl.ds(pl.multiple_of(cpar + kk * HIDDEN, HIDDEN),
                                    HIDDEN)],
                    out_hbm.at[pl.ds(po, HIDDEN)],
                    semouts[par],
                )

        def process_group(g, par):
            g16 = pl.multiple_of(g * 16, 16)
            sv = sidx_v[pl.ds(g16, 16)]
            pv = pos_v[pl.ds(g16, 16)]
            nv = newf_v[pl.ds(g16, 16)]
            dv = did_v[pl.ds(g16, 16)]
            cpar = par * CSTG
            fire(sv, nv, dv, 0)
            fire(sv, nv, dv, 1)
            drain_extract(sv, pv, nv, dv, 0, cpar, par)
            fire(sv, nv, dv, 2)
            drain_extract(sv, pv, nv, dv, 1, cpar, par)
            fire(sv, nv, dv, 3)
            drain_extract(sv, pv, nv, dv, 2, cpar, par)
            drain_extract(sv, pv, nv, dv, 3, cpar, par)

        def drain_out(par):
            pltpu.make_async_copy(
                out_hbm.at[pl.ds(0, CSTG)],
                cstg_v.at[pl.ds(par * CSTG, CSTG)],
                semouts[par],
            ).wait()

        def body(t, carry):
            @pl.when(t >= 1)
            def _():
                drain_out(0)

            process_group(2 * t, 0)

            @pl.when(t >= 1)
            def _():
                drain_out(1)

            process_group(2 * t + 1, 1)
            return carry

        lax.fori_loop(0, GROUPS // 2, body, 0)
        drain_out(0)
        drain_out(1)

    return emb


_EMB = _build()


def kernel(labels, embedding_table):
    idx32 = labels.astype(jnp.int32)
    order = jnp.argsort(idx32).astype(jnp.int32)
    sidx = jnp.take(idx32, order)
    blk = sidx >> 7
    first = (jnp.arange(BATCH, dtype=jnp.int32) % RPW) == 0
    shifted = jnp.concatenate(
        [jnp.ones((1,), jnp.bool_), blk[1:] != blk[:-1]]
    )
    newf = (first | shifted).astype(jnp.int32)
    did = jnp.cumsum(newf).astype(jnp.int32) - 1
    table_t = jnp.swapaxes(embedding_table, 0, 1)
    out1d = _EMB(sidx, order, newf, did, table_t)
    return out1d.reshape(BATCH, HIDDEN)
